# baseline (speedup 1.0000x reference)
import jax
import jax.numpy as jnp
from jax.experimental import pallas as pl


_BL = 512


def _copy_body(t_ref, o_ref):
    for b in range(4):
        o_ref[b] = t_ref[...]


def kernel(x, table):
    B, L, D = x.shape
    n_l = L // _BL
    out = pl.pallas_call(
        _copy_body,
        grid=(n_l,),
        in_specs=[pl.BlockSpec((_BL, D), lambda l: (l, 0))],
        out_specs=pl.BlockSpec((B, _BL, D), lambda l: (0, l, 0)),
        out_shape=jax.ShapeDtypeStruct((B, L, D), table.dtype),
    )(table)
    return out


# speedup vs baseline: 1.0187x; 1.0187x over previous
import jax
import jax.numpy as jnp
from jax.experimental import pallas as pl
from jax.experimental.pallas import tpu as pltpu


_L, _D, _B = 8192, 1024, 4
_CH = 1024
_NCH = _L // _CH


def _read(t_hbm, vbuf, rsem, c):
    return pltpu.make_async_copy(
        t_hbm.at[pl.ds(c * _CH, _CH)],
        vbuf.at[pl.ds(c * _CH, _CH)],
        rsem.at[c],
    )


def _write(o_hbm, vbuf, wsem, c, b):
    return pltpu.make_async_copy(
        vbuf.at[pl.ds(c * _CH, _CH)],
        o_hbm.at[b, pl.ds(c * _CH, _CH)],
        wsem.at[c],
    )


def _body(t_hbm, o_hbm, vbuf, rsem, wsem):
    _read(t_hbm, vbuf, rsem, 0).start()
    for c in range(_NCH):
        _read(t_hbm, vbuf, rsem, c).wait()
        if c + 1 < _NCH:
            _read(t_hbm, vbuf, rsem, c + 1).start()
        for b in range(_B):
            _write(o_hbm, vbuf, wsem, c, b).start()
        if c >= 1:
            for b in range(_B):
                _write(o_hbm, vbuf, wsem, c - 1, b).wait()
    for b in range(_B):
        _write(o_hbm, vbuf, wsem, _NCH - 1, b).wait()


def kernel(x, table):
    B, L, D = x.shape
    out = pl.pallas_call(
        _body,
        in_specs=[pl.BlockSpec(memory_space=pltpu.MemorySpace.HBM)],
        out_specs=pl.BlockSpec(memory_space=pltpu.MemorySpace.HBM),
        out_shape=jax.ShapeDtypeStruct((B, L, D), table.dtype),
        scratch_shapes=[
            pltpu.VMEM((L, D), table.dtype),
            pltpu.SemaphoreType.DMA((_NCH,)),
            pltpu.SemaphoreType.DMA((_NCH,)),
        ],
    )(table)
    return out
